# baseline (speedup 1.0000x reference)
import jax
import jax.numpy as jnp
from jax.experimental import pallas as pl
from jax.experimental.pallas import tpu as pltpu

_NC = 100
_B = 65536
_S = 4
_R = 4096
_G = _B // (_R * _S)


def _body(x0, x1, x2, x3, t0, t1, t2, t3, w_ref, out_ref, acc_ref):
    i = pl.program_id(0)

    @pl.when(i == 0)
    def _():
        acc_ref[...] = jnp.zeros_like(acc_ref)

    w = w_ref[...]
    ones_row = jnp.ones((1, _NC), jnp.float32)
    cls = jax.lax.broadcasted_iota(jnp.int32, (_R, _NC), 1)
    cls128 = jax.lax.broadcasted_iota(jnp.int32, (_R, 128), 1)

    for x_ref, t_ref in ((x0, t0), (x1, t1), (x2, t2), (x3, t3)):
        x = x_ref[...]
        t = t_ref[...]
        e = jnp.exp(x) * w
        e_masked = jnp.where(t == cls, e, 0.0)
        s = jax.lax.dot_general(ones_row, e, (((1,), (1,)), ((), ())),
                                preferred_element_type=jnp.float32)
        et = jax.lax.dot_general(ones_row, e_masked, (((1,), (1,)), ((), ())),
                                 preferred_element_type=jnp.float32)
        p = jnp.clip(et / s, 1e-5, 1.0)

        oh128 = (t == cls128).astype(jnp.float32)
        pstack = jnp.concatenate([p, jnp.ones_like(p)], axis=0)
        part = jax.lax.dot_general(pstack, oh128, (((1,), (0,)), ((), ())),
                                   preferred_element_type=jnp.float32)
        acc_ref[...] += part

    @pl.when(i == _G - 1)
    def _():
        sums = acc_ref[0:1, :]
        counts = acc_ref[1:2, :]
        exist = counts != 0.0
        denom = jnp.where(exist, counts, 1.0)
        meanp = sums / denom
        safe = jnp.where(exist, meanp, 1.0)
        ml = -jnp.log(safe)
        pw = jnp.where(exist, ml * ml * ml, 0.0)
        n_exist = jnp.sum(exist.astype(jnp.float32))
        msum = jnp.sum(pw) / n_exist
        loss = jnp.exp(jnp.log(msum) / 3.0)
        out_ref[...] = jnp.broadcast_to(loss, (1, 1))


def kernel(output, target, weight):
    t2 = target.reshape(_B, 1)
    x_specs = [pl.BlockSpec((_R, _NC), lambda i, k=k: (_S * i + k, 0))
               for k in range(_S)]
    t_specs = [pl.BlockSpec((_R, 1), lambda i, k=k: (_S * i + k, 0))
               for k in range(_S)]
    res = pl.pallas_call(
        _body,
        grid=(_G,),
        in_specs=x_specs + t_specs + [pl.BlockSpec((1, _NC), lambda i: (0, 0))],
        out_specs=pl.BlockSpec((1, 1), lambda i: (0, 0)),
        out_shape=jax.ShapeDtypeStruct((1, 1), jnp.float32),
        scratch_shapes=[pltpu.VMEM((2, 128), jnp.float32)],
        compiler_params=pltpu.CompilerParams(
            dimension_semantics=("arbitrary",)),
    )(output, output, output, output, t2, t2, t2, t2,
      weight.reshape(1, _NC))
    return res[0, 0]
